# baseline (speedup 1.0000x reference)
import functools

import jax
import jax.numpy as jnp
from jax.experimental import pallas as pl
from jax.experimental.pallas import tpu as pltpu
from jax.experimental.pallas import tpu_sc as plsc

_NS = 16
_K = 80
_G = 64


def _pad_nodes(N):
    return ((N + 8 * _NS - 1) // (8 * _NS)) * (8 * _NS)


_P = 5


@functools.partial(jax.jit, static_argnums=(2, 3, 4))
def _sc_agg(x, ei_flat, N, E, D):
    per_tile = E // _NS
    n_chunks = per_tile // _K
    NP = _pad_nodes(N)
    stripe = NP // _NS
    mesh = plsc.VectorSubcoreMesh(core_axis_name="c", subcore_axis_name="s")

    cpp = n_chunks // _P

    @functools.partial(
        pl.kernel,
        out_type=jax.ShapeDtypeStruct((2 * NP, D), jnp.float32),
        mesh=mesh,
        scratch_types=[
            pltpu.VMEM((cpp, _K), jnp.int32),
            pltpu.VMEM((_K,), jnp.int32),
            pltpu.VMEM((_K, D), jnp.float32),
            pltpu.VMEM((_K, D), jnp.float32),
            pltpu.VMEM_SHARED((NP, D), jnp.float32),
            pltpu.SemaphoreType.DMA,
            pltpu.SemaphoreType.DMA,
        ],
    )
    def body(x_hbm, ei_hbm, z_hbm, out_hbm, gidx, sidx, rows0, rows1,
             acc, sem0, sem1):
        c = jax.lax.axis_index("c")
        s = jax.lax.axis_index("s")
        pltpu.sync_copy(z_hbm.at[pl.ds(s * stripe, stripe)],
                        acc.at[pl.ds(s * stripe, stripe)])
        plsc.subcore_barrier()

        rows = (rows0, rows1)
        sems = (sem0, sem1)
        for ph in range(_P):
            grow = (c * _NS + s) * _P + ph
            srow = ((1 - c) * _NS + s) * _P + ph
            pltpu.sync_copy(ei_hbm.at[grow], gidx)
            pltpu.async_copy(x_hbm.at[gidx.at[0]], rows0, sem0)
            pltpu.async_copy(x_hbm.at[gidx.at[1]], rows1, sem1)

            def step(j, carry):
                for b in range(2):
                    i = 2 * j + b
                    pltpu.sync_copy(ei_hbm.at[srow, i], sidx)
                    pltpu.make_async_copy(x_hbm.at[gidx.at[i]], rows[b],
                                          sems[b]).wait()
                    pltpu.sync_copy(rows[b], acc.at[sidx], add=True)
                    pltpu.async_copy(x_hbm.at[gidx.at[i + 2]], rows[b], sems[b])
                return carry

            jax.lax.fori_loop(0, cpp // 2 - 1, step, 0)
            for b in range(2):
                i = cpp - 2 + b
                pltpu.sync_copy(ei_hbm.at[srow, i], sidx)
                pltpu.make_async_copy(x_hbm.at[gidx.at[i]], rows[b],
                                      sems[b]).wait()
                pltpu.sync_copy(rows[b], acc.at[sidx], add=True)

        plsc.subcore_barrier()
        pltpu.sync_copy(acc.at[pl.ds(s * stripe, stripe)],
                        out_hbm.at[pl.ds(c * NP + s * stripe, stripe)])

    zeros = jnp.zeros((NP, D), jnp.float32)
    ei_r = ei_flat.reshape(2 * _NS * _P, cpp, _K)
    return body(x, ei_r, zeros)


@functools.partial(jax.jit, static_argnums=(1, 2))
def _sc_degree(ei_flat, N, E):
    per_tile = E // _NS
    n_chunks = per_tile // _K
    NP = ((N + 16 * _NS - 1) // (16 * _NS)) * (16 * _NS)
    stripe = NP // _NS
    mesh = plsc.VectorSubcoreMesh(core_axis_name="c", subcore_axis_name="s")

    @functools.partial(
        pl.kernel,
        out_type=jax.ShapeDtypeStruct((2 * NP,), jnp.float32),
        mesh=mesh,
        scratch_types=[
            pltpu.VMEM((per_tile,), jnp.int32),
            pltpu.VMEM((_K,), jnp.float32),
            pltpu.VMEM_SHARED((NP,), jnp.float32),
        ],
    )
    def body(ei_hbm, z_hbm, ones_hbm, out_hbm, sidx, ones_v, acc):
        c = jax.lax.axis_index("c")
        s = jax.lax.axis_index("s")
        pltpu.sync_copy(z_hbm.at[pl.ds(s * stripe, stripe)],
                        acc.at[pl.ds(s * stripe, stripe)])
        pltpu.sync_copy(ones_hbm, ones_v)
        pltpu.sync_copy(ei_hbm.at[pl.ds((1 - c) * E + s * per_tile, per_tile)],
                        sidx)
        plsc.subcore_barrier()

        def chunk(i, carry):
            pltpu.sync_copy(ones_v, acc.at[sidx.at[pl.ds(i * _K, _K)]], add=True)
            return carry

        jax.lax.fori_loop(0, n_chunks, chunk, 0)
        plsc.subcore_barrier()
        pltpu.sync_copy(acc.at[pl.ds(s * stripe, stripe)],
                        out_hbm.at[pl.ds(c * NP + s * stripe, stripe)])

    zeros = jnp.zeros((NP,), jnp.float32)
    ones = jnp.ones((_K,), jnp.float32)
    return body(ei_flat, zeros, ones)


def _tc_layer_body(x_ref, aggd_ref, aggs_ref, cntd_ref, cnts_ref,
                   ws_ref, wd_ref, wt_ref, bs_ref, bd_ref, bt_ref, o_ref):
    dn = (((1,), (1,)), ((), ()))
    hi = jax.lax.Precision.HIGHEST
    x = x_ref[...]
    md = aggd_ref[...] / jnp.maximum(cntd_ref[...], 1.0)
    ms = aggs_ref[...] / jnp.maximum(cnts_ref[...], 1.0)
    acc = jax.lax.dot_general(x, ws_ref[...], dn, precision=hi,
                              preferred_element_type=jnp.float32)
    acc = acc + 0.5 * jax.lax.dot_general(md, wd_ref[...], dn, precision=hi,
                                          preferred_element_type=jnp.float32)
    acc = acc + 0.5 * jax.lax.dot_general(ms, wt_ref[...], dn, precision=hi,
                                          preferred_element_type=jnp.float32)
    acc = acc + bs_ref[...] + 0.5 * bd_ref[...] + 0.5 * bt_ref[...]
    o_ref[...] = jnp.maximum(acc, 0.0)


def _tc_layer(x, aggd, aggs, cntd, cnts, Wself, bself, Wstd, bstd, Wdts, bdts):
    N, D = x.shape
    H = Wself.shape[0]
    BR = 1000
    grid = (N // BR,)
    full = lambda shp: pl.BlockSpec(shp, lambda i: (0, 0))
    return pl.pallas_call(
        _tc_layer_body,
        grid=grid,
        in_specs=[
            pl.BlockSpec((BR, D), lambda i: (i, 0)),
            pl.BlockSpec((BR, D), lambda i: (i, 0)),
            pl.BlockSpec((BR, D), lambda i: (i, 0)),
            pl.BlockSpec((BR, 1), lambda i: (i, 0)),
            pl.BlockSpec((BR, 1), lambda i: (i, 0)),
            full((H, D)), full((H, D)), full((H, D)),
            full((1, H)), full((1, H)), full((1, H)),
        ],
        out_specs=pl.BlockSpec((BR, H), lambda i: (i, 0)),
        out_shape=jax.ShapeDtypeStruct((N, H), jnp.float32),
    )(x, aggd, aggs, cntd, cnts, Wself, Wstd, Wdts,
      bself.reshape(1, H), bstd.reshape(1, H), bdts.reshape(1, H))


def _tc_head_body(h_ref, b_ref, w1_ref, b1_ref, w2_ref, b2_ref, o_ref, pool_ref):
    i = pl.program_id(0)

    @pl.when(i == 0)
    def _():
        pool_ref[...] = jnp.full(pool_ref.shape, -jnp.inf, jnp.float32)

    bids = b_ref[...]
    h = h_ref[...]

    def upd(g, carry):
        m = bids == g
        v = jnp.max(jnp.where(m, h, -jnp.inf), axis=0, keepdims=True)
        pool_ref[pl.ds(g, 1), :] = jnp.maximum(pool_ref[pl.ds(g, 1), :], v)
        return carry

    jax.lax.fori_loop(0, _G, upd, 0)

    @pl.when(i == pl.num_programs(0) - 1)
    def _():
        dn = (((1,), (1,)), ((), ()))
        t = jax.lax.dot_general(pool_ref[...], w1_ref[...], dn,
                                precision=jax.lax.Precision.HIGHEST,
                                preferred_element_type=jnp.float32) + b1_ref[...]
        t = jnp.maximum(t, 0.0)
        p = t * w2_ref[...]
        o_ref[...] = jnp.sum(p, axis=1, keepdims=True) + b2_ref[0, 0]


def _tc_head(h, batch, W1, b1, W2, b2):
    N, H = h.shape
    BR = 1000
    grid = (N // BR,)
    batch2 = batch.reshape(N, 1)
    full = lambda shp: pl.BlockSpec(shp, lambda i: tuple(0 for _ in shp))
    return pl.pallas_call(
        _tc_head_body,
        grid=grid,
        in_specs=[
            pl.BlockSpec((BR, H), lambda i: (i, 0)),
            pl.BlockSpec((BR, 1), lambda i: (i, 0)),
            full(W1.shape), full((1, W1.shape[0])),
            full(W2.shape), full((1, 1)),
        ],
        out_specs=pl.BlockSpec((_G, 1), lambda i: (0, 0)),
        out_shape=jax.ShapeDtypeStruct((_G, 1), jnp.float32),
        scratch_shapes=[pltpu.VMEM((_G, H), jnp.float32)],
    )(h, batch2, W1, b1.reshape(1, -1), W2, b2.reshape(1, 1))


def kernel(x, edge_index, batch,
           W_self1, b_self1, W_std1, b_std1, W_dts1, b_dts1,
           W_self2, b_self2, W_std2, b_std2, W_dts2, b_dts2,
           W_self3, b_self3, W_std3, b_std3, W_dts3, b_dts3,
           W_lin1, b_lin1, W_lin2, b_lin2):
    N, D = x.shape
    E = edge_index.shape[1]
    ei_flat = edge_index.reshape(-1)

    NP = _pad_nodes(N)
    NPd = ((N + 16 * _NS - 1) // (16 * _NS)) * (16 * _NS)
    cnt_flat = _sc_degree(ei_flat, N, E)
    cntd = cnt_flat[:N].reshape(N, 1)
    cnts = cnt_flat[NPd:NPd + N].reshape(N, 1)

    h = x
    layers = [
        (W_self1, b_self1, W_std1, b_std1, W_dts1, b_dts1),
        (W_self2, b_self2, W_std2, b_std2, W_dts2, b_dts2),
        (W_self3, b_self3, W_std3, b_std3, W_dts3, b_dts3),
    ]
    for (Ws, bs, Wd, bd, Wt, bt) in layers:
        agg_flat = _sc_agg(h, ei_flat, N, E, D)
        aggd, aggs = agg_flat[:N], agg_flat[NP:NP + N]
        h = _tc_layer(h, aggd, aggs, cntd, cnts, Ws, bs, Wd, bd, Wt, bt)

    return _tc_head(h, batch, W_lin1, b_lin1, W_lin2, b_lin2)


# speedup vs baseline: 1.2470x; 1.2470x over previous
import functools

import jax
import jax.numpy as jnp
from jax.experimental import pallas as pl
from jax.experimental.pallas import tpu as pltpu
from jax.experimental.pallas import tpu_sc as plsc

_NS = 16
_K = 80
_G = 64


def _pad_nodes(N):
    return ((N + 8 * _NS - 1) // (8 * _NS)) * (8 * _NS)


_P = 5


@functools.partial(jax.jit, static_argnums=(2, 3, 4))
def _sc_agg(x, ei_flat, N, E, D):
    per_tile = E // _NS
    n_chunks = per_tile // _K
    NP = _pad_nodes(N)
    stripe = NP // _NS
    mesh = plsc.VectorSubcoreMesh(core_axis_name="c", subcore_axis_name="s")

    cpp = n_chunks // _P

    _NB = 3
    main_iters = (cpp - _NB) // _NB

    @functools.partial(
        pl.kernel,
        out_type=jax.ShapeDtypeStruct((2 * NP, D), jnp.float32),
        mesh=mesh,
        scratch_types=[
            pltpu.VMEM((cpp, _K), jnp.int32),
            pltpu.VMEM((cpp, _K), jnp.int32),
            pltpu.VMEM((_K, D), jnp.float32),
            pltpu.VMEM((_K, D), jnp.float32),
            pltpu.VMEM((_K, D), jnp.float32),
            pltpu.VMEM_SHARED((NP, D), jnp.float32),
            pltpu.SemaphoreType.DMA,
            pltpu.SemaphoreType.DMA,
            pltpu.SemaphoreType.DMA,
        ],
    )
    def body(x_hbm, ei_hbm, z_hbm, out_hbm, gidx, sidx,
             rows0, rows1, rows2, acc, sem0, sem1, sem2):
        c = jax.lax.axis_index("c")
        s = jax.lax.axis_index("s")
        pltpu.sync_copy(z_hbm.at[pl.ds(s * stripe, stripe)],
                        acc.at[pl.ds(s * stripe, stripe)])
        plsc.subcore_barrier()

        rows = (rows0, rows1, rows2)
        sems = (sem0, sem1, sem2)
        for ph in range(_P):
            grow = (c * _NS + s) * _P + ph
            srow = ((1 - c) * _NS + s) * _P + ph
            pltpu.sync_copy(ei_hbm.at[grow], gidx)
            pltpu.sync_copy(ei_hbm.at[srow], sidx)
            for b in range(_NB):
                pltpu.async_copy(x_hbm.at[gidx.at[b]], rows[b], sems[b])

            def step(j, carry):
                for b in range(_NB):
                    i = _NB * j + b
                    pltpu.make_async_copy(x_hbm.at[gidx.at[i]], rows[b],
                                          sems[b]).wait()
                    pltpu.sync_copy(rows[b], acc.at[sidx.at[i]], add=True)
                    pltpu.async_copy(x_hbm.at[gidx.at[i + _NB]], rows[b],
                                     sems[b])
                return carry

            jax.lax.fori_loop(0, main_iters, step, 0)
            for i in range(_NB * main_iters, cpp):
                b = i % _NB
                pltpu.make_async_copy(x_hbm.at[gidx.at[i]], rows[b],
                                      sems[b]).wait()
                pltpu.sync_copy(rows[b], acc.at[sidx.at[i]], add=True)
                if i + _NB < cpp:
                    pltpu.async_copy(x_hbm.at[gidx.at[i + _NB]], rows[b],
                                     sems[b])

        plsc.subcore_barrier()
        pltpu.sync_copy(acc.at[pl.ds(s * stripe, stripe)],
                        out_hbm.at[pl.ds(c * NP + s * stripe, stripe)])

    zeros = jnp.zeros((NP, D), jnp.float32)
    ei_r = ei_flat.reshape(2 * _NS * _P, cpp, _K)
    return body(x, ei_r, zeros)


@functools.partial(jax.jit, static_argnums=(1, 2))
def _sc_degree(ei_flat, N, E):
    per_tile = E // _NS
    n_chunks = per_tile // _K
    NP = ((N + 16 * _NS - 1) // (16 * _NS)) * (16 * _NS)
    stripe = NP // _NS
    mesh = plsc.VectorSubcoreMesh(core_axis_name="c", subcore_axis_name="s")

    @functools.partial(
        pl.kernel,
        out_type=jax.ShapeDtypeStruct((2 * NP,), jnp.float32),
        mesh=mesh,
        scratch_types=[
            pltpu.VMEM((per_tile,), jnp.int32),
            pltpu.VMEM((_K,), jnp.float32),
            pltpu.VMEM_SHARED((NP,), jnp.float32),
        ],
    )
    def body(ei_hbm, z_hbm, ones_hbm, out_hbm, sidx, ones_v, acc):
        c = jax.lax.axis_index("c")
        s = jax.lax.axis_index("s")
        pltpu.sync_copy(z_hbm.at[pl.ds(s * stripe, stripe)],
                        acc.at[pl.ds(s * stripe, stripe)])
        pltpu.sync_copy(ones_hbm, ones_v)
        pltpu.sync_copy(ei_hbm.at[pl.ds((1 - c) * E + s * per_tile, per_tile)],
                        sidx)
        plsc.subcore_barrier()

        def chunk(i, carry):
            pltpu.sync_copy(ones_v, acc.at[sidx.at[pl.ds(i * _K, _K)]], add=True)
            return carry

        jax.lax.fori_loop(0, n_chunks, chunk, 0)
        plsc.subcore_barrier()
        pltpu.sync_copy(acc.at[pl.ds(s * stripe, stripe)],
                        out_hbm.at[pl.ds(c * NP + s * stripe, stripe)])

    zeros = jnp.zeros((NP,), jnp.float32)
    ones = jnp.ones((_K,), jnp.float32)
    return body(ei_flat, zeros, ones)


def _tc_layer_body(x_ref, aggd_ref, aggs_ref, cntd_ref, cnts_ref,
                   ws_ref, wd_ref, wt_ref, bs_ref, bd_ref, bt_ref, o_ref):
    dn = (((1,), (1,)), ((), ()))
    hi = jax.lax.Precision.HIGHEST
    x = x_ref[...]
    md = aggd_ref[...] / jnp.maximum(cntd_ref[...], 1.0)
    ms = aggs_ref[...] / jnp.maximum(cnts_ref[...], 1.0)
    acc = jax.lax.dot_general(x, ws_ref[...], dn, precision=hi,
                              preferred_element_type=jnp.float32)
    acc = acc + 0.5 * jax.lax.dot_general(md, wd_ref[...], dn, precision=hi,
                                          preferred_element_type=jnp.float32)
    acc = acc + 0.5 * jax.lax.dot_general(ms, wt_ref[...], dn, precision=hi,
                                          preferred_element_type=jnp.float32)
    acc = acc + bs_ref[...] + 0.5 * bd_ref[...] + 0.5 * bt_ref[...]
    o_ref[...] = jnp.maximum(acc, 0.0)


def _tc_layer(x, aggd, aggs, cntd, cnts, Wself, bself, Wstd, bstd, Wdts, bdts):
    N, D = x.shape
    H = Wself.shape[0]
    BR = 1000
    grid = (N // BR,)
    full = lambda shp: pl.BlockSpec(shp, lambda i: (0, 0))
    return pl.pallas_call(
        _tc_layer_body,
        grid=grid,
        in_specs=[
            pl.BlockSpec((BR, D), lambda i: (i, 0)),
            pl.BlockSpec((BR, D), lambda i: (i, 0)),
            pl.BlockSpec((BR, D), lambda i: (i, 0)),
            pl.BlockSpec((BR, 1), lambda i: (i, 0)),
            pl.BlockSpec((BR, 1), lambda i: (i, 0)),
            full((H, D)), full((H, D)), full((H, D)),
            full((1, H)), full((1, H)), full((1, H)),
        ],
        out_specs=pl.BlockSpec((BR, H), lambda i: (i, 0)),
        out_shape=jax.ShapeDtypeStruct((N, H), jnp.float32),
    )(x, aggd, aggs, cntd, cnts, Wself, Wstd, Wdts,
      bself.reshape(1, H), bstd.reshape(1, H), bdts.reshape(1, H))


def _tc_head_body(h_ref, b_ref, w1_ref, b1_ref, w2_ref, b2_ref, o_ref, pool_ref):
    i = pl.program_id(0)

    @pl.when(i == 0)
    def _():
        pool_ref[...] = jnp.full(pool_ref.shape, -jnp.inf, jnp.float32)

    bids = b_ref[...]
    h = h_ref[...]

    def upd(g, carry):
        m = bids == g
        v = jnp.max(jnp.where(m, h, -jnp.inf), axis=0, keepdims=True)
        pool_ref[pl.ds(g, 1), :] = jnp.maximum(pool_ref[pl.ds(g, 1), :], v)
        return carry

    jax.lax.fori_loop(0, _G, upd, 0)

    @pl.when(i == pl.num_programs(0) - 1)
    def _():
        dn = (((1,), (1,)), ((), ()))
        t = jax.lax.dot_general(pool_ref[...], w1_ref[...], dn,
                                precision=jax.lax.Precision.HIGHEST,
                                preferred_element_type=jnp.float32) + b1_ref[...]
        t = jnp.maximum(t, 0.0)
        p = t * w2_ref[...]
        o_ref[...] = jnp.sum(p, axis=1, keepdims=True) + b2_ref[0, 0]


def _tc_head(h, batch, W1, b1, W2, b2):
    N, H = h.shape
    BR = 1000
    grid = (N // BR,)
    batch2 = batch.reshape(N, 1)
    full = lambda shp: pl.BlockSpec(shp, lambda i: tuple(0 for _ in shp))
    return pl.pallas_call(
        _tc_head_body,
        grid=grid,
        in_specs=[
            pl.BlockSpec((BR, H), lambda i: (i, 0)),
            pl.BlockSpec((BR, 1), lambda i: (i, 0)),
            full(W1.shape), full((1, W1.shape[0])),
            full(W2.shape), full((1, 1)),
        ],
        out_specs=pl.BlockSpec((_G, 1), lambda i: (0, 0)),
        out_shape=jax.ShapeDtypeStruct((_G, 1), jnp.float32),
        scratch_shapes=[pltpu.VMEM((_G, H), jnp.float32)],
    )(h, batch2, W1, b1.reshape(1, -1), W2, b2.reshape(1, 1))


def kernel(x, edge_index, batch,
           W_self1, b_self1, W_std1, b_std1, W_dts1, b_dts1,
           W_self2, b_self2, W_std2, b_std2, W_dts2, b_dts2,
           W_self3, b_self3, W_std3, b_std3, W_dts3, b_dts3,
           W_lin1, b_lin1, W_lin2, b_lin2):
    N, D = x.shape
    E = edge_index.shape[1]
    ei_flat = edge_index.reshape(-1)

    NP = _pad_nodes(N)
    NPd = ((N + 16 * _NS - 1) // (16 * _NS)) * (16 * _NS)
    cnt_flat = _sc_degree(ei_flat, N, E)
    cntd = cnt_flat[:N].reshape(N, 1)
    cnts = cnt_flat[NPd:NPd + N].reshape(N, 1)

    h = x
    layers = [
        (W_self1, b_self1, W_std1, b_std1, W_dts1, b_dts1),
        (W_self2, b_self2, W_std2, b_std2, W_dts2, b_dts2),
        (W_self3, b_self3, W_std3, b_std3, W_dts3, b_dts3),
    ]
    for (Ws, bs, Wd, bd, Wt, bt) in layers:
        agg_flat = _sc_agg(h, ei_flat, N, E, D)
        aggd, aggs = agg_flat[:N], agg_flat[NP:NP + N]
        h = _tc_layer(h, aggd, aggs, cntd, cnts, Ws, bs, Wd, bd, Wt, bt)

    return _tc_head(h, batch, W_lin1, b_lin1, W_lin2, b_lin2)


# speedup vs baseline: 1.2680x; 1.0168x over previous
import functools

import jax
import jax.numpy as jnp
from jax.experimental import pallas as pl
from jax.experimental.pallas import tpu as pltpu
from jax.experimental.pallas import tpu_sc as plsc

_NS = 16
_K = 80
_G = 64


def _pad_nodes(N):
    return ((N + 8 * _NS - 1) // (8 * _NS)) * (8 * _NS)


_P = 5


@functools.partial(jax.jit, static_argnums=(2, 3, 4, 5))
def _sc_agg(x, ei_flat, N, E, D, with_deg=False):
    per_tile = E // _NS
    n_chunks = per_tile // _K
    NP = _pad_nodes(N)
    NPd = ((N + 16 * _NS - 1) // (16 * _NS)) * (16 * _NS)
    striped = NPd // _NS
    stripe = NP // _NS
    mesh = plsc.VectorSubcoreMesh(core_axis_name="c", subcore_axis_name="s")

    cpp = n_chunks // _P

    _NB = 3
    main_iters = (cpp - _NB) // _NB

    out_type = [jax.ShapeDtypeStruct((2 * NP, D), jnp.float32)]
    scratch = [
        pltpu.VMEM((cpp, _K), jnp.int32),
        pltpu.VMEM((cpp, _K), jnp.int32),
        pltpu.VMEM((_K, D), jnp.float32),
        pltpu.VMEM((_K, D), jnp.float32),
        pltpu.VMEM((_K, D), jnp.float32),
        pltpu.VMEM_SHARED((NP, D), jnp.float32),
    ]
    if with_deg:
        out_type.append(jax.ShapeDtypeStruct((2 * NPd,), jnp.float32))
        scratch.append(pltpu.VMEM((_K,), jnp.float32))
        scratch.append(pltpu.VMEM_SHARED((NPd,), jnp.float32))
    scratch += [pltpu.SemaphoreType.DMA] * _NB

    @functools.partial(pl.kernel, mesh=mesh, scratch_types=scratch,
                       out_type=tuple(out_type) if with_deg else out_type[0])
    def body(*refs):
        if with_deg:
            (x_hbm, ei_hbm, z_hbm, z1_hbm, ones_hbm, out_hbm, dout_hbm,
             gidx, sidx, rows0, rows1, rows2, acc, ones_v, dacc,
             sem0, sem1, sem2) = refs
        else:
            (x_hbm, ei_hbm, z_hbm, out_hbm, gidx, sidx, rows0, rows1, rows2,
             acc, sem0, sem1, sem2) = refs
        c = jax.lax.axis_index("c")
        s = jax.lax.axis_index("s")
        pltpu.sync_copy(z_hbm.at[pl.ds(s * stripe, stripe)],
                        acc.at[pl.ds(s * stripe, stripe)])
        if with_deg:
            pltpu.sync_copy(z1_hbm.at[pl.ds(s * striped, striped)],
                            dacc.at[pl.ds(s * striped, striped)])
            pltpu.sync_copy(ones_hbm, ones_v)
        plsc.subcore_barrier()

        rows = (rows0, rows1, rows2)
        sems = (sem0, sem1, sem2)

        def chunk_step(i, b):
            pltpu.make_async_copy(x_hbm.at[gidx.at[i]], rows[b],
                                  sems[b]).wait()
            pltpu.sync_copy(rows[b], acc.at[sidx.at[i]], add=True)
            if with_deg:
                pltpu.sync_copy(ones_v, dacc.at[sidx.at[i]], add=True)

        for ph in range(_P):
            grow = (c * _NS + s) * _P + ph
            srow = ((1 - c) * _NS + s) * _P + ph
            pltpu.sync_copy(ei_hbm.at[grow], gidx)
            pltpu.sync_copy(ei_hbm.at[srow], sidx)
            for b in range(_NB):
                pltpu.async_copy(x_hbm.at[gidx.at[b]], rows[b], sems[b])

            def step(j, carry):
                for b in range(_NB):
                    i = _NB * j + b
                    chunk_step(i, b)
                    pltpu.async_copy(x_hbm.at[gidx.at[i + _NB]], rows[b],
                                     sems[b])
                return carry

            jax.lax.fori_loop(0, main_iters, step, 0)
            for i in range(_NB * main_iters, cpp):
                b = i % _NB
                chunk_step(i, b)
                if i + _NB < cpp:
                    pltpu.async_copy(x_hbm.at[gidx.at[i + _NB]], rows[b],
                                     sems[b])

        plsc.subcore_barrier()
        pltpu.sync_copy(acc.at[pl.ds(s * stripe, stripe)],
                        out_hbm.at[pl.ds(c * NP + s * stripe, stripe)])
        if with_deg:
            pltpu.sync_copy(dacc.at[pl.ds(s * striped, striped)],
                            dout_hbm.at[pl.ds(c * NPd + s * striped, striped)])

    zeros = jnp.zeros((NP, D), jnp.float32)
    ei_r = ei_flat.reshape(2 * _NS * _P, cpp, _K)
    if with_deg:
        return body(x, ei_r, zeros, jnp.zeros((NPd,), jnp.float32),
                    jnp.ones((_K,), jnp.float32))
    return body(x, ei_r, zeros)


def _tc_layer_body(x_ref, aggd_ref, aggs_ref, cntd_ref, cnts_ref,
                   ws_ref, wd_ref, wt_ref, bs_ref, bd_ref, bt_ref, o_ref):
    dn = (((1,), (1,)), ((), ()))
    hi = jax.lax.Precision.HIGHEST
    x = x_ref[...]
    md = aggd_ref[...] / jnp.maximum(cntd_ref[...], 1.0)
    ms = aggs_ref[...] / jnp.maximum(cnts_ref[...], 1.0)
    acc = jax.lax.dot_general(x, ws_ref[...], dn, precision=hi,
                              preferred_element_type=jnp.float32)
    acc = acc + 0.5 * jax.lax.dot_general(md, wd_ref[...], dn, precision=hi,
                                          preferred_element_type=jnp.float32)
    acc = acc + 0.5 * jax.lax.dot_general(ms, wt_ref[...], dn, precision=hi,
                                          preferred_element_type=jnp.float32)
    acc = acc + bs_ref[...] + 0.5 * bd_ref[...] + 0.5 * bt_ref[...]
    o_ref[...] = jnp.maximum(acc, 0.0)


def _tc_layer(x, aggd, aggs, cntd, cnts, Wself, bself, Wstd, bstd, Wdts, bdts):
    N, D = x.shape
    H = Wself.shape[0]
    BR = 1000
    grid = (N // BR,)
    full = lambda shp: pl.BlockSpec(shp, lambda i: (0, 0))
    return pl.pallas_call(
        _tc_layer_body,
        grid=grid,
        in_specs=[
            pl.BlockSpec((BR, D), lambda i: (i, 0)),
            pl.BlockSpec((BR, D), lambda i: (i, 0)),
            pl.BlockSpec((BR, D), lambda i: (i, 0)),
            pl.BlockSpec((BR, 1), lambda i: (i, 0)),
            pl.BlockSpec((BR, 1), lambda i: (i, 0)),
            full((H, D)), full((H, D)), full((H, D)),
            full((1, H)), full((1, H)), full((1, H)),
        ],
        out_specs=pl.BlockSpec((BR, H), lambda i: (i, 0)),
        out_shape=jax.ShapeDtypeStruct((N, H), jnp.float32),
    )(x, aggd, aggs, cntd, cnts, Wself, Wstd, Wdts,
      bself.reshape(1, H), bstd.reshape(1, H), bdts.reshape(1, H))


def _tc_layer3_head_body(x_ref, aggd_ref, aggs_ref, cntd_ref, cnts_ref,
                         ws_ref, wd_ref, wt_ref, bs_ref, bd_ref, bt_ref,
                         b_ref, w1_ref, b1_ref, w2_ref, b2_ref,
                         o_ref, pool_ref):
    i = pl.program_id(0)

    @pl.when(i == 0)
    def _():
        pool_ref[...] = jnp.full(pool_ref.shape, -jnp.inf, jnp.float32)

    dn = (((1,), (1,)), ((), ()))
    hi = jax.lax.Precision.HIGHEST
    x = x_ref[...]
    md = aggd_ref[...] / jnp.maximum(cntd_ref[...], 1.0)
    ms = aggs_ref[...] / jnp.maximum(cnts_ref[...], 1.0)
    acc = jax.lax.dot_general(x, ws_ref[...], dn, precision=hi,
                              preferred_element_type=jnp.float32)
    acc = acc + 0.5 * jax.lax.dot_general(md, wd_ref[...], dn, precision=hi,
                                          preferred_element_type=jnp.float32)
    acc = acc + 0.5 * jax.lax.dot_general(ms, wt_ref[...], dn, precision=hi,
                                          preferred_element_type=jnp.float32)
    acc = acc + bs_ref[...] + 0.5 * bd_ref[...] + 0.5 * bt_ref[...]
    h = jnp.maximum(acc, 0.0)

    bids = b_ref[...]

    def upd(g, carry):
        m = bids == g
        v = jnp.max(jnp.where(m, h, -jnp.inf), axis=0, keepdims=True)
        pool_ref[pl.ds(g, 1), :] = jnp.maximum(pool_ref[pl.ds(g, 1), :], v)
        return carry

    jax.lax.fori_loop(0, _G, upd, 0)

    @pl.when(i == pl.num_programs(0) - 1)
    def _():
        t = jax.lax.dot_general(pool_ref[...], w1_ref[...], dn,
                                precision=jax.lax.Precision.HIGHEST,
                                preferred_element_type=jnp.float32) + b1_ref[...]
        t = jnp.maximum(t, 0.0)
        p = t * w2_ref[...]
        o_ref[...] = jnp.sum(p, axis=1, keepdims=True) + b2_ref[0, 0]


def _tc_layer3_head(x, aggd, aggs, cntd, cnts, Wself, bself, Wstd, bstd,
                    Wdts, bdts, batch, W1, b1, W2, b2):
    N, D = x.shape
    H = Wself.shape[0]
    BR = 1000
    grid = (N // BR,)
    batch2 = batch.reshape(N, 1)
    full = lambda shp: pl.BlockSpec(shp, lambda i: tuple(0 for _ in shp))
    return pl.pallas_call(
        _tc_layer3_head_body,
        grid=grid,
        in_specs=[
            pl.BlockSpec((BR, D), lambda i: (i, 0)),
            pl.BlockSpec((BR, D), lambda i: (i, 0)),
            pl.BlockSpec((BR, D), lambda i: (i, 0)),
            pl.BlockSpec((BR, 1), lambda i: (i, 0)),
            pl.BlockSpec((BR, 1), lambda i: (i, 0)),
            full((H, D)), full((H, D)), full((H, D)),
            full((1, H)), full((1, H)), full((1, H)),
            pl.BlockSpec((BR, 1), lambda i: (i, 0)),
            full(W1.shape), full((1, W1.shape[0])),
            full(W2.shape), full((1, 1)),
        ],
        out_specs=pl.BlockSpec((_G, 1), lambda i: (0, 0)),
        out_shape=jax.ShapeDtypeStruct((_G, 1), jnp.float32),
        scratch_shapes=[pltpu.VMEM((_G, H), jnp.float32)],
    )(x, aggd, aggs, cntd, cnts, Wself, Wstd, Wdts,
      bself.reshape(1, H), bstd.reshape(1, H), bdts.reshape(1, H),
      batch2, W1, b1.reshape(1, -1), W2, b2.reshape(1, 1))


def kernel(x, edge_index, batch,
           W_self1, b_self1, W_std1, b_std1, W_dts1, b_dts1,
           W_self2, b_self2, W_std2, b_std2, W_dts2, b_dts2,
           W_self3, b_self3, W_std3, b_std3, W_dts3, b_dts3,
           W_lin1, b_lin1, W_lin2, b_lin2):
    N, D = x.shape
    E = edge_index.shape[1]
    ei_flat = edge_index.reshape(-1)

    NP = _pad_nodes(N)
    NPd = ((N + 16 * _NS - 1) // (16 * _NS)) * (16 * _NS)

    agg_flat, cnt_flat = _sc_agg(x, ei_flat, N, E, D, True)
    cntd = cnt_flat[:N].reshape(N, 1)
    cnts = cnt_flat[NPd:NPd + N].reshape(N, 1)
    aggd, aggs = agg_flat[:N], agg_flat[NP:NP + N]
    h = _tc_layer(x, aggd, aggs, cntd, cnts, W_self1, b_self1, W_std1, b_std1,
                  W_dts1, b_dts1)

    agg_flat = _sc_agg(h, ei_flat, N, E, D)
    aggd, aggs = agg_flat[:N], agg_flat[NP:NP + N]
    h = _tc_layer(h, aggd, aggs, cntd, cnts, W_self2, b_self2, W_std2, b_std2,
                  W_dts2, b_dts2)

    agg_flat = _sc_agg(h, ei_flat, N, E, D)
    aggd, aggs = agg_flat[:N], agg_flat[NP:NP + N]
    return _tc_layer3_head(h, aggd, aggs, cntd, cnts, W_self3, b_self3,
                           W_std3, b_std3, W_dts3, b_dts3,
                           batch, W_lin1, b_lin1, W_lin2, b_lin2)
